# Initial kernel scaffold; baseline (speedup 1.0000x reference)
import jax
import jax.numpy as jnp
from jax.experimental import pallas as pl


def kernel(H_support, H_query, y_support, support_mask, ln_gamma, ln_beta, W_Q, W_K, tau_param, bias):
    raise NotImplementedError("write your pallas kernel here")


# speedup vs baseline: 15.8319x; 15.8319x over previous
import jax
import jax.numpy as jnp
from jax.experimental import pallas as pl
from jax.experimental.pallas import tpu as pltpu

_TOPK = 64
_CPAD = 16
_MBLK = 256


def _body(hq_ref, hs_ref, y_ref, msk_ref, g_ref, b_ref, wq_ref, wk_ref, sc_ref,
          out_ref, k_scr, oh_scr):
    mb = pl.program_id(1)
    eps = jnp.float32(1e-5)
    tau = sc_ref[0:1, 0:1]
    bias = sc_ref[0:1, 1:2]

    @pl.when(mb == 0)
    def _prep():
        hs = hs_ref[0]
        mu = jnp.mean(hs, axis=-1, keepdims=True)
        var = jnp.mean((hs - mu) * (hs - mu), axis=-1, keepdims=True)
        hsn = (hs - mu) * jax.lax.rsqrt(var + eps) * g_ref[...] + b_ref[...]
        k_scr[...] = jax.lax.dot_general(
            hsn, wk_ref[...], (((1,), (1,)), ((), ())),
            preferred_element_type=jnp.float32)
        y2 = y_ref[0]
        cls = jax.lax.broadcasted_iota(jnp.int32, (_CPAD, 1), 0)
        oh_scr[...] = (y2 == cls).astype(jnp.float32) * msk_ref[0]

    hq = hq_ref[0]
    mu = jnp.mean(hq, axis=-1, keepdims=True)
    var = jnp.mean((hq - mu) * (hq - mu), axis=-1, keepdims=True)
    hqn = (hq - mu) * jax.lax.rsqrt(var + eps) * g_ref[...] + b_ref[...]
    q = jax.lax.dot_general(hqn, wq_ref[...], (((1,), (1,)), ((), ())),
                            preferred_element_type=jnp.float32)
    logits = jax.lax.dot_general(q, k_scr[...], (((1,), (1,)), ((), ())),
                                 preferred_element_type=jnp.float32)
    logits = logits * tau + bias
    logits = jnp.where(msk_ref[0] > 0.5, logits, jnp.float32(-1e30))

    bits = jax.lax.bitcast_convert_type(logits, jnp.int32)
    skey = jnp.where(bits < 0, bits ^ jnp.int32(0x7FFFFFFF), bits)

    minint = jnp.int32(-2147483648)

    def step(i, u):
        bit = jnp.left_shift(jnp.int32(1), jnp.int32(31) - i)
        cand_u = u | bit
        cand_s = cand_u ^ minint
        cnt = jnp.sum((skey >= cand_s).astype(jnp.float32), axis=1,
                      keepdims=True)
        return jnp.where(cnt >= jnp.float32(_TOPK), cand_u, u)

    u0 = jnp.zeros((logits.shape[0], 1), jnp.int32)
    u = jax.lax.fori_loop(0, 32, step, u0)
    thr = u ^ minint
    keep = skey >= thr

    x = jnp.where(keep, logits, jnp.float32(-1e4))
    sig = 1.0 / (1.0 + jnp.exp(-x))
    p = jax.lax.dot_general(sig, oh_scr[...], (((1,), (1,)), ((), ())),
                            preferred_element_type=jnp.float32)
    s = jnp.sum(p, axis=1, keepdims=True)
    p = p / jnp.maximum(s, jnp.float32(1e-12))
    p = jnp.maximum(p, jnp.float32(1e-12))
    out_ref[...] = jnp.log(p).reshape(1, p.shape[0], _CPAD)


def kernel(H_support, H_query, y_support, support_mask, ln_gamma, ln_beta,
           W_Q, W_K, tau_param, bias):
    B, N, D = H_support.shape
    M = H_query.shape[1]
    tau = jax.nn.softplus(tau_param) + jnp.float32(1e-6)
    sc = jnp.zeros((1, 128), jnp.float32).at[0, 0].set(tau).at[0, 1].set(bias[0])
    y3 = y_support.reshape(B, 1, N).astype(jnp.int32)
    m3 = support_mask.reshape(B, 1, N).astype(jnp.float32)
    g2 = ln_gamma.reshape(1, D)
    b2 = ln_beta.reshape(1, D)

    grid = (B, M // _MBLK)
    out = pl.pallas_call(
        _body,
        grid=grid,
        in_specs=[
            pl.BlockSpec((1, _MBLK, D), lambda b, mb: (b, mb, 0)),
            pl.BlockSpec((1, N, D), lambda b, mb: (b, 0, 0)),
            pl.BlockSpec((1, 1, N), lambda b, mb: (b, 0, 0)),
            pl.BlockSpec((1, 1, N), lambda b, mb: (b, 0, 0)),
            pl.BlockSpec((1, D), lambda b, mb: (0, 0)),
            pl.BlockSpec((1, D), lambda b, mb: (0, 0)),
            pl.BlockSpec((D, D), lambda b, mb: (0, 0)),
            pl.BlockSpec((D, D), lambda b, mb: (0, 0)),
            pl.BlockSpec((1, 128), lambda b, mb: (0, 0)),
        ],
        out_specs=pl.BlockSpec((1, _MBLK, _CPAD), lambda b, mb: (b, mb, 0)),
        out_shape=jax.ShapeDtypeStruct((B, M, _CPAD), jnp.float32),
        scratch_shapes=[
            pltpu.VMEM((N, D), jnp.float32),
            pltpu.VMEM((_CPAD, N), jnp.float32),
        ],
        compiler_params=pltpu.CompilerParams(
            dimension_semantics=("arbitrary", "arbitrary")),
    )(H_query, H_support, y3, m3, g2, b2, W_Q, W_K, sc)
    return out[..., :10]


# speedup vs baseline: 22.1051x; 1.3962x over previous
import jax
import jax.numpy as jnp
from jax.experimental import pallas as pl
from jax.experimental.pallas import tpu as pltpu

_TOPK = 64
_CPAD = 16
_MBLK = 256
_BISECT = 20


def _body(hq_ref, hs_ref, y_ref, msk_ref, g_ref, b_ref, wq_ref, wk_ref, sc_ref,
          out_ref, k_scr, oh_scr):
    mb = pl.program_id(1)
    eps = jnp.float32(1e-5)
    tau = sc_ref[0:1, 0:1]
    bias = sc_ref[0:1, 1:2]

    @pl.when(mb == 0)
    def _prep():
        hs = hs_ref[0]
        mu = jnp.mean(hs, axis=-1, keepdims=True)
        var = jnp.mean((hs - mu) * (hs - mu), axis=-1, keepdims=True)
        hsn = (hs - mu) * jax.lax.rsqrt(var + eps) * g_ref[...] + b_ref[...]
        k_scr[...] = jax.lax.dot_general(
            hsn, wk_ref[...], (((1,), (1,)), ((), ())),
            preferred_element_type=jnp.float32)
        y2 = y_ref[0]
        cls = jax.lax.broadcasted_iota(jnp.int32, (_CPAD, 1), 0)
        oh_scr[...] = (y2 == cls).astype(jnp.float32) * msk_ref[0]

    hq = hq_ref[0]
    mu = jnp.mean(hq, axis=-1, keepdims=True)
    var = jnp.mean((hq - mu) * (hq - mu), axis=-1, keepdims=True)
    hqn = (hq - mu) * jax.lax.rsqrt(var + eps) * g_ref[...] + b_ref[...]
    q = jax.lax.dot_general(hqn, wq_ref[...], (((1,), (1,)), ((), ())),
                            preferred_element_type=jnp.float32)
    logits = jax.lax.dot_general(q, k_scr[...], (((1,), (1,)), ((), ())),
                                 preferred_element_type=jnp.float32)
    logits = logits * tau + bias
    logits = jnp.where(msk_ref[0] > 0.5, logits, jnp.float32(-1e30))

    n = logits.shape[1]
    t = logits
    while t.shape[1] > _TOPK:
        h = t.shape[1] // 2
        t = jnp.maximum(t[:, :h], t[:, h:])
    lo = jnp.min(t, axis=1, keepdims=True)
    hi = jnp.max(t, axis=1, keepdims=True)

    def step(i, c):
        lo, hi = c
        mid = 0.5 * (lo + hi)
        cnt = jnp.sum((logits >= mid).astype(jnp.float32), axis=1,
                      keepdims=True)
        big = cnt >= jnp.float32(_TOPK)
        return jnp.where(big, mid, lo), jnp.where(big, hi, mid)

    lo, hi = jax.lax.fori_loop(0, _BISECT, step, (lo, hi))
    keep = logits >= lo

    x = jnp.where(keep, logits, jnp.float32(-1e4))
    sig = 1.0 / (1.0 + jnp.exp(-x))
    p = jax.lax.dot_general(sig, oh_scr[...], (((1,), (1,)), ((), ())),
                            preferred_element_type=jnp.float32)
    s = jnp.sum(p, axis=1, keepdims=True)
    p = p / jnp.maximum(s, jnp.float32(1e-12))
    p = jnp.maximum(p, jnp.float32(1e-12))
    out_ref[...] = jnp.log(p).reshape(1, p.shape[0], _CPAD)


def kernel(H_support, H_query, y_support, support_mask, ln_gamma, ln_beta,
           W_Q, W_K, tau_param, bias):
    B, N, D = H_support.shape
    M = H_query.shape[1]
    tau = jax.nn.softplus(tau_param) + jnp.float32(1e-6)
    sc = jnp.zeros((1, 128), jnp.float32).at[0, 0].set(tau).at[0, 1].set(bias[0])
    y3 = y_support.reshape(B, 1, N).astype(jnp.int32)
    m3 = support_mask.reshape(B, 1, N).astype(jnp.float32)
    g2 = ln_gamma.reshape(1, D)
    b2 = ln_beta.reshape(1, D)

    grid = (B, M // _MBLK)
    out = pl.pallas_call(
        _body,
        grid=grid,
        in_specs=[
            pl.BlockSpec((1, _MBLK, D), lambda b, mb: (b, mb, 0)),
            pl.BlockSpec((1, N, D), lambda b, mb: (b, 0, 0)),
            pl.BlockSpec((1, 1, N), lambda b, mb: (b, 0, 0)),
            pl.BlockSpec((1, 1, N), lambda b, mb: (b, 0, 0)),
            pl.BlockSpec((1, D), lambda b, mb: (0, 0)),
            pl.BlockSpec((1, D), lambda b, mb: (0, 0)),
            pl.BlockSpec((D, D), lambda b, mb: (0, 0)),
            pl.BlockSpec((D, D), lambda b, mb: (0, 0)),
            pl.BlockSpec((1, 128), lambda b, mb: (0, 0)),
        ],
        out_specs=pl.BlockSpec((1, _MBLK, _CPAD), lambda b, mb: (b, mb, 0)),
        out_shape=jax.ShapeDtypeStruct((B, M, _CPAD), jnp.float32),
        scratch_shapes=[
            pltpu.VMEM((N, D), jnp.float32),
            pltpu.VMEM((_CPAD, N), jnp.float32),
        ],
        compiler_params=pltpu.CompilerParams(
            dimension_semantics=("arbitrary", "arbitrary")),
    )(H_query, H_support, y3, m3, g2, b2, W_Q, W_K, sc)
    return out[..., :10]


# speedup vs baseline: 23.8077x; 1.0770x over previous
import jax
import jax.numpy as jnp
from jax.experimental import pallas as pl
from jax.experimental.pallas import tpu as pltpu

_TOPK = 64
_CPAD = 16
_MBLK = 512
_BISECT = 20


def _body(hq_ref, hs_ref, y_ref, msk_ref, g_ref, b_ref, wq_ref, wk_ref, sc_ref,
          out_ref, k_scr, oh_scr):
    mb = pl.program_id(1)
    eps = jnp.float32(1e-5)
    tau = sc_ref[0:1, 0:1]
    bias = sc_ref[0:1, 1:2]

    @pl.when(mb == 0)
    def _prep():
        hs = hs_ref[0]
        mu = jnp.mean(hs, axis=-1, keepdims=True)
        var = jnp.mean((hs - mu) * (hs - mu), axis=-1, keepdims=True)
        hsn = (hs - mu) * jax.lax.rsqrt(var + eps) * g_ref[...] + b_ref[...]
        k_scr[...] = jax.lax.dot_general(
            hsn, wk_ref[...], (((1,), (1,)), ((), ())),
            preferred_element_type=jnp.float32)
        y2 = y_ref[0]
        cls = jax.lax.broadcasted_iota(jnp.int32, (_CPAD, 1), 0)
        oh_scr[...] = (y2 == cls).astype(jnp.float32) * msk_ref[0]

    hq = hq_ref[0]
    mu = jnp.mean(hq, axis=-1, keepdims=True)
    var = jnp.mean((hq - mu) * (hq - mu), axis=-1, keepdims=True)
    hqn = (hq - mu) * jax.lax.rsqrt(var + eps) * g_ref[...] + b_ref[...]
    q = jax.lax.dot_general(hqn, wq_ref[...], (((1,), (1,)), ((), ())),
                            preferred_element_type=jnp.float32)
    logits = jax.lax.dot_general(q, k_scr[...], (((1,), (1,)), ((), ())),
                                 preferred_element_type=jnp.float32)
    logits = logits * tau + bias
    logits = jnp.where(msk_ref[0] > 0.5, logits, jnp.float32(-1e30))

    n = logits.shape[1]
    t = logits
    while t.shape[1] > _TOPK:
        h = t.shape[1] // 2
        t = jnp.maximum(t[:, :h], t[:, h:])
    lo = jnp.min(t, axis=1, keepdims=True)
    hi = jnp.max(t, axis=1, keepdims=True)

    def step(i, c):
        lo, hi = c
        mid = 0.5 * (lo + hi)
        cnt = jnp.sum((logits >= mid).astype(jnp.float32), axis=1,
                      keepdims=True)
        big = cnt >= jnp.float32(_TOPK)
        return jnp.where(big, mid, lo), jnp.where(big, hi, mid)

    lo, hi = jax.lax.fori_loop(0, _BISECT, step, (lo, hi))
    keep = logits >= lo

    x = jnp.where(keep, logits, jnp.float32(-1e4))
    sig = 1.0 / (1.0 + jnp.exp(-x))
    p = jax.lax.dot_general(sig, oh_scr[...], (((1,), (1,)), ((), ())),
                            preferred_element_type=jnp.float32)
    s = jnp.sum(p, axis=1, keepdims=True)
    p = p / jnp.maximum(s, jnp.float32(1e-12))
    p = jnp.maximum(p, jnp.float32(1e-12))
    out_ref[...] = jnp.log(p).reshape(1, p.shape[0], _CPAD)


def kernel(H_support, H_query, y_support, support_mask, ln_gamma, ln_beta,
           W_Q, W_K, tau_param, bias):
    B, N, D = H_support.shape
    M = H_query.shape[1]
    tau = jax.nn.softplus(tau_param) + jnp.float32(1e-6)
    sc = jnp.zeros((1, 128), jnp.float32).at[0, 0].set(tau).at[0, 1].set(bias[0])
    y3 = y_support.reshape(B, 1, N).astype(jnp.int32)
    m3 = support_mask.reshape(B, 1, N).astype(jnp.float32)
    g2 = ln_gamma.reshape(1, D)
    b2 = ln_beta.reshape(1, D)

    grid = (B, M // _MBLK)
    out = pl.pallas_call(
        _body,
        grid=grid,
        in_specs=[
            pl.BlockSpec((1, _MBLK, D), lambda b, mb: (b, mb, 0)),
            pl.BlockSpec((1, N, D), lambda b, mb: (b, 0, 0)),
            pl.BlockSpec((1, 1, N), lambda b, mb: (b, 0, 0)),
            pl.BlockSpec((1, 1, N), lambda b, mb: (b, 0, 0)),
            pl.BlockSpec((1, D), lambda b, mb: (0, 0)),
            pl.BlockSpec((1, D), lambda b, mb: (0, 0)),
            pl.BlockSpec((D, D), lambda b, mb: (0, 0)),
            pl.BlockSpec((D, D), lambda b, mb: (0, 0)),
            pl.BlockSpec((1, 128), lambda b, mb: (0, 0)),
        ],
        out_specs=pl.BlockSpec((1, _MBLK, _CPAD), lambda b, mb: (b, mb, 0)),
        out_shape=jax.ShapeDtypeStruct((B, M, _CPAD), jnp.float32),
        scratch_shapes=[
            pltpu.VMEM((N, D), jnp.float32),
            pltpu.VMEM((_CPAD, N), jnp.float32),
        ],
        compiler_params=pltpu.CompilerParams(
            dimension_semantics=("arbitrary", "arbitrary")),
    )(H_query, H_support, y3, m3, g2, b2, W_Q, W_K, sc)
    return out[..., :10]
